# baseline (speedup 1.0000x reference)
import jax
import jax.numpy as jnp
from jax import lax
from jax.experimental import pallas as pl
from jax.experimental.pallas import tpu as pltpu
from jax.experimental.pallas import tpu_sc as plsc

D = 32
SCALE = D ** 0.5
NC_ = 2
NS_ = 16
NW = NC_ * NS_
JT = 25
BR = 1024


def _sc_embed(idx4_hbm, table_hbm, out5_hbm, idx_v, rows_v, tbuf, gsem, osem):
    w = lax.axis_index("s") * NC_ + lax.axis_index("c")

    def stage_and_fire(jt, buf):
        pltpu.sync_copy(idx4_hbm.at[jt, w], idx_v.at[buf])
        for ji in range(8):
            pltpu.async_copy(table_hbm.at[idx_v.at[buf, ji]],
                             rows_v.at[buf, pl.ds(ji * 128, 128)], gsem.at[buf])

    def wait_out():
        pltpu.make_async_copy(tbuf, out5_hbm.at[pl.ds(0, 8), :, w],
                              osem).wait()

    def process(jt, buf, after_drain=None):
        for _ in range(8):
            pltpu.make_async_copy(table_hbm.at[idx_v.at[buf, 0]],
                                  rows_v.at[buf, pl.ds(0, 128)],
                                  gsem.at[buf]).wait()
        if after_drain is not None:
            after_drain()
        rows2 = rows_v.at[buf]
        vi = lax.iota(jnp.int32, 16)
        for ji in range(8):
            rowvecs = [vi + (ji * 128 + c * 16) for c in range(8)]

            def col_body(i, carry, rowvecs=rowvecs, ji=ji):
                colv = jnp.full((16,), i, dtype=jnp.int32)
                ut = i // 8
                ui = i % 8
                for c in range(8):
                    v = plsc.load_gather(rows2, [rowvecs[c], colv])
                    tbuf[ji, ut, ui, pl.ds(c * 16, 16)] = v * SCALE
                return carry

            lax.fori_loop(0, 4 * 8, col_body, 0)
        pltpu.async_copy(tbuf, out5_hbm.at[pl.ds(jt * 8, 8), :, w], osem)

    stage_and_fire(0, 0)

    def pair_body(p, carry):
        a = 2 * p
        stage_and_fire(a + 1, 1)
        process(a, 0, after_drain=lambda: pl.when(p >= 1)(wait_out))
        stage_and_fire(a + 2, 0)
        process(a + 1, 1, after_drain=wait_out)
        return carry

    lax.fori_loop(0, (JT - 1) // 2, pair_body, 0)
    process(JT - 1, 0, after_drain=wait_out)
    wait_out()


def kernel(inputs, lookup_table):
    idx4 = (inputs.astype(jnp.int32)
            .reshape(32, 128, JT, 8)
            .transpose(2, 0, 3, 1))
    mesh = plsc.VectorSubcoreMesh(core_axis_name="c", subcore_axis_name="s")
    out5 = pl.kernel(
        _sc_embed,
        out_type=jax.ShapeDtypeStruct((200, 4, 32, 8, 128), jnp.float32),
        mesh=mesh,
        compiler_params=pltpu.CompilerParams(use_tc_tiling_on_sc=False,
                                             needs_layout_passes=False),
        scratch_types=[
            pltpu.VMEM((2, 8, 128), jnp.int32),
            pltpu.VMEM((2, BR, D), jnp.float32),
            pltpu.VMEM((8, 4, 8, 128), jnp.float32),
            pltpu.SemaphoreType.DMA((2,)),
            pltpu.SemaphoreType.DMA,
        ],
    )(idx4, lookup_table)
    return out5.transpose(2, 4, 0, 1, 3).reshape(4096, 200, D)
